# Initial kernel scaffold; baseline (speedup 1.0000x reference)
import functools

import jax
import jax.numpy as jnp
from jax import lax
from jax.experimental import pallas as pl
from jax.experimental.pallas import tpu as pltpu
from jax.experimental.pallas import tpu_sc as plsc

N = 10000
D = 128
H = 128
E = 320000

RW = 144
NC = 2
NS = 16
NW = NC * NS
CHUNK = 128
NCHUNK = 79
T_TILE = CHUNK * NCHUNK
E_PAD = NW * T_TILE
NP = 10016
ROWS_PER_TILE = NP // NS
ZB = 64

BLK = 400
GRID = N // BLK


def _ln(h, g, b):
    m = jnp.mean(h, axis=-1, keepdims=True)
    v = jnp.mean((h - m) * (h - m), axis=-1, keepdims=True)
    return (h - m) / jnp.sqrt(v + 1e-5) * g + b


def _proj_body(x_ref, nt_ref, pg, pbta, pwt, pbi, fg, fb, fwt, fbi,
               sg, sb, swt, sbi, emb_ref, out_ref):
    x = jnp.clip(x_ref[...], -10.0, 10.0)
    p = jnp.dot(_ln(x, pg[...], pbta[...]), pwt[...],
                preferred_element_type=jnp.float32) + pbi[...]
    f = jnp.dot(_ln(x, fg[...], fb[...]), fwt[...],
                preferred_element_type=jnp.float32) + fbi[...]
    s = jnp.dot(_ln(x, sg[...], sb[...]), swt[...],
                preferred_element_type=jnp.float32) + sbi[...]
    nt = nt_ref[...]
    sel = jnp.where(nt == 0, p, jnp.where(nt == 1, f,
                    jnp.where(nt == 2, s, 0.0)))
    te = jnp.where(nt == 0, emb_ref[0:1, :], jnp.where(
        nt == 1, emb_ref[1:2, :], emb_ref[2:3, :]))
    h = sel + te
    out_ref[:, 0:128] = h
    lane = lax.broadcasted_iota(jnp.int32, (BLK, RW - 128), 1)
    out_ref[:, 128:RW] = jnp.where(lane == 0, 1.0, 0.0)


def _proj(x, nt2, pg, pbta, pwt, pbi, fg, fb, fwt, fbi, sg, sb, swt, sbi, emb):
    row = lambda i: (i, 0)
    full = lambda i: (0, 0)
    vec = pl.BlockSpec((1, H), full)
    return pl.pallas_call(
        _proj_body,
        grid=(GRID,),
        in_specs=[
            pl.BlockSpec((BLK, D), row),
            pl.BlockSpec((BLK, 1), row),
            vec, vec, pl.BlockSpec((D, H), full), vec,
            vec, vec, pl.BlockSpec((D, H), full), vec,
            vec, vec, pl.BlockSpec((D, H), full), vec,
            pl.BlockSpec((8, H), full),
        ],
        out_specs=pl.BlockSpec((BLK, RW), row),
        out_shape=jax.ShapeDtypeStruct((N, RW), jnp.float32),
    )(x, nt2, pg, pbta, pwt, pbi, fg, fb, fwt, fbi, sg, sb, swt, sbi, emb)


def _agg_body(h_hbm, src_hbm, dst_hbm, out_hbm,
              src_v, dst_v, rows0, rows1, zb_v, acc_sh, sem0, sem1):
    c = lax.axis_index("c")
    s = lax.axis_index("s")
    wid = c * NS + s

    pltpu.sync_copy(src_hbm.at[wid], src_v)
    pltpu.sync_copy(dst_hbm.at[wid], dst_v)

    def zrow(r, _):
        def zcol(j, _):
            zb_v[r, pl.ds(j * 16, 16)] = jnp.zeros((16,), jnp.float32)
            return 0
        return lax.fori_loop(0, RW // 16, zcol, 0)
    lax.fori_loop(0, ZB, zrow, 0)

    base = s * ROWS_PER_TILE
    def zacc(i, _):
        pltpu.sync_copy(zb_v, acc_sh.at[pl.ds(base + i * ZB, ZB)])
        return 0
    nfull = ROWS_PER_TILE // ZB
    lax.fori_loop(0, nfull, zacc, 0)
    rem = ROWS_PER_TILE - nfull * ZB
    if rem:
        pltpu.sync_copy(zb_v.at[pl.ds(0, rem)],
                        acc_sh.at[pl.ds(base + nfull * ZB, rem)])
    plsc.subcore_barrier()

    def start_gather(i, buf, sem):
        pltpu.async_copy(h_hbm.at[src_v.at[i]], buf, sem)

    start_gather(0, rows0, sem0)
    start_gather(1, rows1, sem1)

    def body(k, _):
        for b, (buf, sem) in enumerate(((rows0, sem0), (rows1, sem1))):
            i = 2 * k + b
            @pl.when(i < NCHUNK)
            def _():
                pltpu.make_async_copy(h_hbm.at[src_v.at[i]], buf, sem).wait()
                pltpu.sync_copy(buf, acc_sh.at[dst_v.at[i]], add=True)
                @pl.when(i + 2 < NCHUNK)
                def _():
                    start_gather(i + 2, buf, sem)
        return 0
    lax.fori_loop(0, (NCHUNK + 1) // 2, body, 0)

    plsc.subcore_barrier()
    pltpu.sync_copy(acc_sh.at[pl.ds(base, ROWS_PER_TILE)],
                    out_hbm.at[c, pl.ds(base, ROWS_PER_TILE)])


def _aggregate(h_aug, src3, dst3):
    mesh = plsc.VectorSubcoreMesh(core_axis_name="c", subcore_axis_name="s",
                                  num_cores=NC, num_subcores=NS)
    return pl.kernel(
        _agg_body,
        out_type=jax.ShapeDtypeStruct((NC, NP, RW), jnp.float32),
        mesh=mesh,
        scratch_types=[
            pltpu.VMEM((NCHUNK, CHUNK), jnp.int32),
            pltpu.VMEM((NCHUNK, CHUNK), jnp.int32),
            pltpu.VMEM((CHUNK, RW), jnp.float32),
            pltpu.VMEM((CHUNK, RW), jnp.float32),
            pltpu.VMEM((ZB, RW), jnp.float32),
            pltpu.VMEM_SHARED((NP, RW), jnp.float32),
            pltpu.SemaphoreType.DMA,
            pltpu.SemaphoreType.DMA,
        ],
    )(h_aug, src3, dst3)


def _layer_body(parts_ref, ha_ref, wlt, bl, wrt, g, b, out_ref, *, final):
    pa = parts_ref[0]
    pb = parts_ref[1]
    sums = pa[:, 0:128] + pb[:, 0:128]
    cnt = pa[:, 128:129] + pb[:, 128:129]
    agg = sums / jnp.maximum(cnt, 1.0)
    h = ha_ref[:, 0:128]
    t = (jnp.dot(agg, wlt[...], preferred_element_type=jnp.float32)
         + jnp.dot(h, wrt[...], preferred_element_type=jnp.float32)
         + bl[...] + h)
    t = jax.nn.relu(_ln(t, g[...], b[...]))
    if final:
        out_ref[...] = t
    else:
        out_ref[:, 0:128] = t
        lane = lax.broadcasted_iota(jnp.int32, (BLK, RW - 128), 1)
        out_ref[:, 128:RW] = jnp.where(lane == 0, 1.0, 0.0)


def _layer(parts, h_aug, wlt, bl, wrt, g, b, final):
    row = lambda i: (i, 0)
    full = lambda i: (0, 0)
    vec = pl.BlockSpec((1, H), full)
    ow = H if final else RW
    return pl.pallas_call(
        functools.partial(_layer_body, final=final),
        grid=(GRID,),
        in_specs=[
            pl.BlockSpec((NC, BLK, RW), lambda i: (0, i, 0)),
            pl.BlockSpec((BLK, RW), row),
            pl.BlockSpec((H, H), full), vec,
            pl.BlockSpec((H, H), full), vec, vec,
        ],
        out_specs=pl.BlockSpec((BLK, ow), row),
        out_shape=jax.ShapeDtypeStruct((N, ow), jnp.float32),
    )(parts, h_aug, wlt, bl, wrt, g, b)


def kernel(x, edge_index, node_type,
           proc_ln_g, proc_ln_b, proc_w, proc_b,
           file_ln_g, file_ln_b, file_w, file_b,
           sock_ln_g, sock_ln_b, sock_w, sock_b,
           type_emb,
           w_l0, b_l0, w_r0, ln_g0, ln_b0,
           w_l1, b_l1, w_r1, ln_g1, ln_b1):
    f32 = jnp.float32
    nt2 = node_type.reshape(N, 1).astype(jnp.int32)
    emb = jnp.zeros((8, H), f32).at[0:3].set(type_emb)
    r1 = lambda v: v.reshape(1, -1).astype(f32)

    h0a = _proj(x, nt2,
                r1(proc_ln_g), r1(proc_ln_b), proc_w.T, r1(proc_b),
                r1(file_ln_g), r1(file_ln_b), file_w.T, r1(file_b),
                r1(sock_ln_g), r1(sock_ln_b), sock_w.T, r1(sock_b),
                emb)

    src = edge_index[0].astype(jnp.int32)
    dst = edge_index[1].astype(jnp.int32)
    pad = E_PAD - E
    src3 = jnp.concatenate([src, jnp.zeros((pad,), jnp.int32)]
                           ).reshape(NW, NCHUNK, CHUNK)
    dst3 = jnp.concatenate([dst, jnp.full((pad,), N, jnp.int32)]
                           ).reshape(NW, NCHUNK, CHUNK)

    parts0 = _aggregate(h0a, src3, dst3)
    h1a = _layer(parts0, h0a, w_l0.T, r1(b_l0), w_r0.T,
                 r1(ln_g0), r1(ln_b0), final=False)
    parts1 = _aggregate(h1a, src3, dst3)
    h2 = _layer(parts1, h1a, w_l1.T, r1(b_l1), w_r1.T,
                r1(ln_g1), r1(ln_b1), final=True)
    return h2


# speedup vs baseline: 4.7561x; 4.7561x over previous
import functools

import jax
import jax.numpy as jnp
from jax import lax
from jax.experimental import pallas as pl
from jax.experimental.pallas import tpu as pltpu
from jax.experimental.pallas import tpu_sc as plsc

N = 10000
D = 128
H = 128
E = 320000

RW = 144
NC = 2
NS = 16
NW = NC * NS
CHUNK = 128
NCHUNK = 79
T_TILE = CHUNK * NCHUNK
E_PAD = NW * T_TILE
NP = 10016
ROWS_PER_TILE = NP // NS
ZB = 64

BLK = 400
GRID = N // BLK


def _ln(h, g, b):
    m = jnp.mean(h, axis=-1, keepdims=True)
    v = jnp.mean((h - m) * (h - m), axis=-1, keepdims=True)
    return (h - m) / jnp.sqrt(v + 1e-5) * g + b


def _proj_body(x_ref, nt_ref, pg, pbta, pwt, pbi, fg, fb, fwt, fbi,
               sg, sb, swt, sbi, emb_ref, out_ref):
    x = jnp.clip(x_ref[...], -10.0, 10.0)
    p = jnp.dot(_ln(x, pg[...], pbta[...]), pwt[...],
                preferred_element_type=jnp.float32) + pbi[...]
    f = jnp.dot(_ln(x, fg[...], fb[...]), fwt[...],
                preferred_element_type=jnp.float32) + fbi[...]
    s = jnp.dot(_ln(x, sg[...], sb[...]), swt[...],
                preferred_element_type=jnp.float32) + sbi[...]
    nt = nt_ref[...]
    sel = jnp.where(nt == 0, p, jnp.where(nt == 1, f,
                    jnp.where(nt == 2, s, 0.0)))
    te = jnp.where(nt == 0, emb_ref[0:1, :], jnp.where(
        nt == 1, emb_ref[1:2, :], emb_ref[2:3, :]))
    h = sel + te
    out_ref[:, 0:128] = h
    lane = lax.broadcasted_iota(jnp.int32, (BLK, RW - 128), 1)
    out_ref[:, 128:RW] = jnp.where(lane == 0, 1.0, 0.0)


def _proj(x, nt2, pg, pbta, pwt, pbi, fg, fb, fwt, fbi, sg, sb, swt, sbi, emb):
    row = lambda i: (i, 0)
    full = lambda i: (0, 0)
    vec = pl.BlockSpec((1, H), full)
    return pl.pallas_call(
        _proj_body,
        grid=(GRID,),
        in_specs=[
            pl.BlockSpec((BLK, D), row),
            pl.BlockSpec((BLK, 1), row),
            vec, vec, pl.BlockSpec((D, H), full), vec,
            vec, vec, pl.BlockSpec((D, H), full), vec,
            vec, vec, pl.BlockSpec((D, H), full), vec,
            pl.BlockSpec((8, H), full),
        ],
        out_specs=pl.BlockSpec((BLK, RW), row),
        out_shape=jax.ShapeDtypeStruct((N, RW), jnp.float32),
    )(x, nt2, pg, pbta, pwt, pbi, fg, fb, fwt, fbi, sg, sb, swt, sbi, emb)


def _agg_body(h_hbm, ed_hbm, zero_hbm, out_hbm,
              idx0, idx1, idx2, idx3, rows0, rows1, acc_sh,
              semg0, semg1, semi0, semi1, semi2, semi3):
    c = lax.axis_index("c")
    s = lax.axis_index("s")
    wid = c * NS + s
    slots = (idx0, idx1, idx2, idx3)
    isems = (semi0, semi1, semi2, semi3)
    bufs = (rows0, rows1)
    gsems = (semg0, semg1)

    base = s * ROWS_PER_TILE
    pltpu.sync_copy(zero_hbm.at[pl.ds(0, ROWS_PER_TILE)],
                    acc_sh.at[pl.ds(base, ROWS_PER_TILE)])
    plsc.subcore_barrier()

    def start_idx(i, q):
        pltpu.async_copy(ed_hbm.at[wid, i], slots[q], isems[q])

    def start_gather(i, q, b):
        pltpu.async_copy(h_hbm.at[slots[q].at[0]], bufs[b], gsems[b])

    pltpu.sync_copy(ed_hbm.at[wid, 0], idx0)
    pltpu.sync_copy(ed_hbm.at[wid, 1], idx1)
    start_gather(0, 0, 0)
    start_gather(1, 1, 1)
    start_idx(2, 2)
    start_idx(3, 3)

    def body(k, _):
        for b4 in range(4):
            i = 4 * k + b4
            q = b4
            b = b4 % 2
            @pl.when(i < NCHUNK)
            def _():
                pltpu.make_async_copy(h_hbm.at[slots[q].at[0]],
                                      bufs[b], gsems[b]).wait()
                pltpu.sync_copy(bufs[b], acc_sh.at[slots[q].at[1]], add=True)
                @pl.when(i + 4 < NCHUNK)
                def _():
                    start_idx(i + 4, q)
                @pl.when(i + 2 < NCHUNK)
                def _():
                    q2 = (b4 + 2) % 4
                    pltpu.make_async_copy(ed_hbm.at[wid, i],
                                          slots[q2], isems[q2]).wait()
                    start_gather(i + 2, q2, b)
        return 0
    lax.fori_loop(0, (NCHUNK + 3) // 4, body, 0)

    plsc.subcore_barrier()
    pltpu.sync_copy(acc_sh.at[pl.ds(base, ROWS_PER_TILE)],
                    out_hbm.at[c, pl.ds(base, ROWS_PER_TILE)])


def _aggregate(h_aug, ed4, zrows):
    mesh = plsc.VectorSubcoreMesh(core_axis_name="c", subcore_axis_name="s",
                                  num_cores=NC, num_subcores=NS)
    return pl.kernel(
        _agg_body,
        out_type=jax.ShapeDtypeStruct((NC, NP, RW), jnp.float32),
        mesh=mesh,
        compiler_params=pltpu.CompilerParams(use_tc_tiling_on_sc=False),
        scratch_types=[
            pltpu.VMEM((2, CHUNK), jnp.int32),
            pltpu.VMEM((2, CHUNK), jnp.int32),
            pltpu.VMEM((2, CHUNK), jnp.int32),
            pltpu.VMEM((2, CHUNK), jnp.int32),
            pltpu.VMEM((CHUNK, RW), jnp.float32),
            pltpu.VMEM((CHUNK, RW), jnp.float32),
            pltpu.VMEM_SHARED((NP, RW), jnp.float32),
            pltpu.SemaphoreType.DMA,
            pltpu.SemaphoreType.DMA,
            pltpu.SemaphoreType.DMA,
            pltpu.SemaphoreType.DMA,
            pltpu.SemaphoreType.DMA,
            pltpu.SemaphoreType.DMA,
        ],
    )(h_aug, ed4, zrows)


def _layer_body(parts_ref, ha_ref, wlt, bl, wrt, g, b, out_ref, *, final):
    pa = parts_ref[0]
    pb = parts_ref[1]
    sums = pa[:, 0:128] + pb[:, 0:128]
    cnt = pa[:, 128:129] + pb[:, 128:129]
    agg = sums / jnp.maximum(cnt, 1.0)
    h = ha_ref[:, 0:128]
    t = (jnp.dot(agg, wlt[...], preferred_element_type=jnp.float32)
         + jnp.dot(h, wrt[...], preferred_element_type=jnp.float32)
         + bl[...] + h)
    t = jax.nn.relu(_ln(t, g[...], b[...]))
    if final:
        out_ref[...] = t
    else:
        out_ref[:, 0:128] = t
        lane = lax.broadcasted_iota(jnp.int32, (BLK, RW - 128), 1)
        out_ref[:, 128:RW] = jnp.where(lane == 0, 1.0, 0.0)


def _layer(parts, h_aug, wlt, bl, wrt, g, b, final):
    row = lambda i: (i, 0)
    full = lambda i: (0, 0)
    vec = pl.BlockSpec((1, H), full)
    ow = H if final else RW
    return pl.pallas_call(
        functools.partial(_layer_body, final=final),
        grid=(GRID,),
        in_specs=[
            pl.BlockSpec((NC, BLK, RW), lambda i: (0, i, 0)),
            pl.BlockSpec((BLK, RW), row),
            pl.BlockSpec((H, H), full), vec,
            pl.BlockSpec((H, H), full), vec, vec,
        ],
        out_specs=pl.BlockSpec((BLK, ow), row),
        out_shape=jax.ShapeDtypeStruct((N, ow), jnp.float32),
    )(parts, h_aug, wlt, bl, wrt, g, b)


def kernel(x, edge_index, node_type,
           proc_ln_g, proc_ln_b, proc_w, proc_b,
           file_ln_g, file_ln_b, file_w, file_b,
           sock_ln_g, sock_ln_b, sock_w, sock_b,
           type_emb,
           w_l0, b_l0, w_r0, ln_g0, ln_b0,
           w_l1, b_l1, w_r1, ln_g1, ln_b1):
    f32 = jnp.float32
    nt2 = node_type.reshape(N, 1).astype(jnp.int32)
    emb = jnp.zeros((8, H), f32).at[0:3].set(type_emb)
    r1 = lambda v: v.reshape(1, -1).astype(f32)

    h0a = _proj(x, nt2,
                r1(proc_ln_g), r1(proc_ln_b), proc_w.T, r1(proc_b),
                r1(file_ln_g), r1(file_ln_b), file_w.T, r1(file_b),
                r1(sock_ln_g), r1(sock_ln_b), sock_w.T, r1(sock_b),
                emb)

    src = edge_index[0].astype(jnp.int32)
    dst = edge_index[1].astype(jnp.int32)
    pad = E_PAD - E
    src3 = jnp.concatenate([src, jnp.zeros((pad,), jnp.int32)]
                           ).reshape(NW, NCHUNK, CHUNK)
    dst3 = jnp.concatenate([dst, jnp.full((pad,), N, jnp.int32)]
                           ).reshape(NW, NCHUNK, CHUNK)
    ed4 = jnp.stack([src3, dst3], axis=2)
    zrows = jnp.zeros((ROWS_PER_TILE, RW), f32)

    parts0 = _aggregate(h0a, ed4, zrows)
    h1a = _layer(parts0, h0a, w_l0.T, r1(b_l0), w_r0.T,
                 r1(ln_g0), r1(ln_b0), final=False)
    parts1 = _aggregate(h1a, ed4, zrows)
    h2 = _layer(parts1, h1a, w_l1.T, r1(b_l1), w_r1.T,
                r1(ln_g1), r1(ln_b1), final=True)
    return h2


# speedup vs baseline: 6.9726x; 1.4661x over previous
import functools

import jax
import jax.numpy as jnp
from jax import lax
from jax.experimental import pallas as pl
from jax.experimental.pallas import tpu as pltpu
from jax.experimental.pallas import tpu_sc as plsc

N = 10000
D = 128
H = 128
E = 320000

RW = 144
NC = 2
NS = 16
NW = NC * NS
CHUNK = 128
N0_CH = 111
N1_CH = 46
TOTCH = NS * (N0_CH + N1_CH)
E_PAD = TOTCH * CHUNK
NP = 10016
ROWS_PER_TILE = NP // NS
ZB = 64

BLK = 400
GRID = N // BLK


def _ln(h, g, b):
    m = jnp.mean(h, axis=-1, keepdims=True)
    v = jnp.mean((h - m) * (h - m), axis=-1, keepdims=True)
    return (h - m) / jnp.sqrt(v + 1e-5) * g + b


def _proj_body(x_ref, nt_ref, pg, pbta, pwt, pbi, fg, fb, fwt, fbi,
               sg, sb, swt, sbi, emb_ref, out_ref):
    x = jnp.clip(x_ref[...], -10.0, 10.0)
    p = jnp.dot(_ln(x, pg[...], pbta[...]), pwt[...],
                preferred_element_type=jnp.float32) + pbi[...]
    f = jnp.dot(_ln(x, fg[...], fb[...]), fwt[...],
                preferred_element_type=jnp.float32) + fbi[...]
    s = jnp.dot(_ln(x, sg[...], sb[...]), swt[...],
                preferred_element_type=jnp.float32) + sbi[...]
    nt = nt_ref[...]
    sel = jnp.where(nt == 0, p, jnp.where(nt == 1, f,
                    jnp.where(nt == 2, s, 0.0)))
    te = jnp.where(nt == 0, emb_ref[0:1, :], jnp.where(
        nt == 1, emb_ref[1:2, :], emb_ref[2:3, :]))
    h = sel + te
    out_ref[:, 0:128] = h
    lane = lax.broadcasted_iota(jnp.int32, (BLK, RW - 128), 1)
    out_ref[:, 128:RW] = jnp.where(lane == 0, 1.0, 0.0)


def _proj(x, nt2, pg, pbta, pwt, pbi, fg, fb, fwt, fbi, sg, sb, swt, sbi, emb):
    row = lambda i: (i, 0)
    full = lambda i: (0, 0)
    vec = pl.BlockSpec((1, H), full)
    return pl.pallas_call(
        _proj_body,
        grid=(GRID,),
        in_specs=[
            pl.BlockSpec((BLK, D), row),
            pl.BlockSpec((BLK, 1), row),
            vec, vec, pl.BlockSpec((D, H), full), vec,
            vec, vec, pl.BlockSpec((D, H), full), vec,
            vec, vec, pl.BlockSpec((D, H), full), vec,
            pl.BlockSpec((8, H), full),
        ],
        out_specs=pl.BlockSpec((BLK, RW), row),
        out_shape=jax.ShapeDtypeStruct((N, RW), jnp.float32),
    )(x, nt2, pg, pbta, pwt, pbi, fg, fb, fwt, fbi, sg, sb, swt, sbi, emb)


def _agg_body(h_hbm, ed_hbm, zero_hbm, out_hbm,
              idx0, idx1, idx2, idx3, rows0, rows1, acc_sh,
              semg0, semg1, semi0, semi1, semi2, semi3):
    c = lax.axis_index("c")
    s = lax.axis_index("s")
    nch = jnp.where(c == 0, N0_CH, N1_CH)
    cbase = c * NS * N0_CH + s * nch
    slots = (idx0, idx1, idx2, idx3)
    isems = (semi0, semi1, semi2, semi3)
    bufs = (rows0, rows1)
    gsems = (semg0, semg1)

    base = s * ROWS_PER_TILE
    pltpu.sync_copy(zero_hbm.at[pl.ds(0, ROWS_PER_TILE)],
                    acc_sh.at[pl.ds(base, ROWS_PER_TILE)])
    plsc.subcore_barrier()

    def start_idx(i, q):
        pltpu.async_copy(ed_hbm.at[cbase + i], slots[q], isems[q])

    def start_gather(i, q, b):
        pltpu.async_copy(h_hbm.at[slots[q].at[0]], bufs[b], gsems[b])

    pltpu.sync_copy(ed_hbm.at[cbase + 0], idx0)
    pltpu.sync_copy(ed_hbm.at[cbase + 1], idx1)
    start_gather(0, 0, 0)
    start_gather(1, 1, 1)
    start_idx(2, 2)
    start_idx(3, 3)

    def body(k, _):
        for b4 in range(4):
            i = 4 * k + b4
            q = b4
            b = b4 % 2
            @pl.when(i < nch)
            def _():
                pltpu.make_async_copy(h_hbm.at[slots[q].at[0]],
                                      bufs[b], gsems[b]).wait()
                pltpu.sync_copy(bufs[b], acc_sh.at[slots[q].at[1]], add=True)
                @pl.when(i + 4 < nch)
                def _():
                    start_idx(i + 4, q)
                @pl.when(i + 2 < nch)
                def _():
                    q2 = (b4 + 2) % 4
                    pltpu.make_async_copy(ed_hbm.at[cbase + i],
                                          slots[q2], isems[q2]).wait()
                    start_gather(i + 2, q2, b)
        return 0
    lax.fori_loop(0, (nch + 3) // 4, body, 0)

    plsc.subcore_barrier()
    pltpu.sync_copy(acc_sh.at[pl.ds(base, ROWS_PER_TILE)],
                    out_hbm.at[c, pl.ds(base, ROWS_PER_TILE)])


def _aggregate(h_aug, ed4, zrows):
    mesh = plsc.VectorSubcoreMesh(core_axis_name="c", subcore_axis_name="s",
                                  num_cores=NC, num_subcores=NS)
    return pl.kernel(
        _agg_body,
        out_type=jax.ShapeDtypeStruct((NC, NP, RW), jnp.float32),
        mesh=mesh,
        compiler_params=pltpu.CompilerParams(use_tc_tiling_on_sc=False),
        scratch_types=[
            pltpu.VMEM((2, CHUNK), jnp.int32),
            pltpu.VMEM((2, CHUNK), jnp.int32),
            pltpu.VMEM((2, CHUNK), jnp.int32),
            pltpu.VMEM((2, CHUNK), jnp.int32),
            pltpu.VMEM((CHUNK, RW), jnp.float32),
            pltpu.VMEM((CHUNK, RW), jnp.float32),
            pltpu.VMEM_SHARED((NP, RW), jnp.float32),
            pltpu.SemaphoreType.DMA,
            pltpu.SemaphoreType.DMA,
            pltpu.SemaphoreType.DMA,
            pltpu.SemaphoreType.DMA,
            pltpu.SemaphoreType.DMA,
            pltpu.SemaphoreType.DMA,
        ],
    )(h_aug, ed4, zrows)


def _layer_body(parts_ref, ha_ref, wlt, bl, wrt, g, b, out_ref, *, final):
    pa = parts_ref[0]
    pb = parts_ref[1]
    sums = pa[:, 0:128] + pb[:, 0:128]
    cnt = pa[:, 128:129] + pb[:, 128:129]
    agg = sums / jnp.maximum(cnt, 1.0)
    h = ha_ref[:, 0:128]
    t = (jnp.dot(agg, wlt[...], preferred_element_type=jnp.float32)
         + jnp.dot(h, wrt[...], preferred_element_type=jnp.float32)
         + bl[...] + h)
    t = jax.nn.relu(_ln(t, g[...], b[...]))
    if final:
        out_ref[...] = t
    else:
        out_ref[:, 0:128] = t
        lane = lax.broadcasted_iota(jnp.int32, (BLK, RW - 128), 1)
        out_ref[:, 128:RW] = jnp.where(lane == 0, 1.0, 0.0)


def _layer(parts, h_aug, wlt, bl, wrt, g, b, final):
    row = lambda i: (i, 0)
    full = lambda i: (0, 0)
    vec = pl.BlockSpec((1, H), full)
    ow = H if final else RW
    return pl.pallas_call(
        functools.partial(_layer_body, final=final),
        grid=(GRID,),
        in_specs=[
            pl.BlockSpec((NC, BLK, RW), lambda i: (0, i, 0)),
            pl.BlockSpec((BLK, RW), row),
            pl.BlockSpec((H, H), full), vec,
            pl.BlockSpec((H, H), full), vec, vec,
        ],
        out_specs=pl.BlockSpec((BLK, ow), row),
        out_shape=jax.ShapeDtypeStruct((N, ow), jnp.float32),
    )(parts, h_aug, wlt, bl, wrt, g, b)


def kernel(x, edge_index, node_type,
           proc_ln_g, proc_ln_b, proc_w, proc_b,
           file_ln_g, file_ln_b, file_w, file_b,
           sock_ln_g, sock_ln_b, sock_w, sock_b,
           type_emb,
           w_l0, b_l0, w_r0, ln_g0, ln_b0,
           w_l1, b_l1, w_r1, ln_g1, ln_b1):
    f32 = jnp.float32
    nt2 = node_type.reshape(N, 1).astype(jnp.int32)
    emb = jnp.zeros((8, H), f32).at[0:3].set(type_emb)
    r1 = lambda v: v.reshape(1, -1).astype(f32)

    h0a = _proj(x, nt2,
                r1(proc_ln_g), r1(proc_ln_b), proc_w.T, r1(proc_b),
                r1(file_ln_g), r1(file_ln_b), file_w.T, r1(file_b),
                r1(sock_ln_g), r1(sock_ln_b), sock_w.T, r1(sock_b),
                emb)

    src = edge_index[0].astype(jnp.int32)
    dst = edge_index[1].astype(jnp.int32)
    pad = E_PAD - E
    src3 = jnp.concatenate([src, jnp.zeros((pad,), jnp.int32)]
                           ).reshape(TOTCH, CHUNK)
    dst3 = jnp.concatenate([dst, jnp.full((pad,), N, jnp.int32)]
                           ).reshape(TOTCH, CHUNK)
    ed4 = jnp.stack([src3, dst3], axis=1)
    zrows = jnp.zeros((ROWS_PER_TILE, RW), f32)

    parts0 = _aggregate(h0a, ed4, zrows)
    h1a = _layer(parts0, h0a, w_l0.T, r1(b_l0), w_r0.T,
                 r1(ln_g0), r1(ln_b0), final=False)
    parts1 = _aggregate(h1a, ed4, zrows)
    h2 = _layer(parts1, h1a, w_l1.T, r1(b_l1), w_r1.T,
                r1(ln_g1), r1(ln_b1), final=True)
    return h2


# speedup vs baseline: 7.6820x; 1.1017x over previous
import functools

import jax
import jax.numpy as jnp
from jax import lax
from jax.experimental import pallas as pl
from jax.experimental.pallas import tpu as pltpu
from jax.experimental.pallas import tpu_sc as plsc

N = 10000
D = 128
H = 128
E = 320000

RW = 160
DT = jnp.bfloat16
NC = 2
NS = 16
NW = NC * NS
CHUNK = 128
N0_CH = 111
N1_CH = 46
TOTCH = NS * (N0_CH + N1_CH)
E_PAD = TOTCH * CHUNK
NP = 10016
ROWS_PER_TILE = NP // NS
ZB = 64

BLK = 400
GRID = N // BLK


def _ln(h, g, b):
    m = jnp.mean(h, axis=-1, keepdims=True)
    v = jnp.mean((h - m) * (h - m), axis=-1, keepdims=True)
    return (h - m) / jnp.sqrt(v + 1e-5) * g + b


def _proj_body(x_ref, nt_ref, pg, pbta, pwt, pbi, fg, fb, fwt, fbi,
               sg, sb, swt, sbi, emb_ref, out_ref):
    x = jnp.clip(x_ref[...], -10.0, 10.0)
    p = jnp.dot(_ln(x, pg[...], pbta[...]), pwt[...],
                preferred_element_type=jnp.float32) + pbi[...]
    f = jnp.dot(_ln(x, fg[...], fb[...]), fwt[...],
                preferred_element_type=jnp.float32) + fbi[...]
    s = jnp.dot(_ln(x, sg[...], sb[...]), swt[...],
                preferred_element_type=jnp.float32) + sbi[...]
    nt = nt_ref[...]
    sel = jnp.where(nt == 0, p, jnp.where(nt == 1, f,
                    jnp.where(nt == 2, s, 0.0)))
    te = jnp.where(nt == 0, emb_ref[0:1, :], jnp.where(
        nt == 1, emb_ref[1:2, :], emb_ref[2:3, :]))
    h = sel + te
    out_ref[:, 0:128] = h.astype(DT)
    lane = lax.broadcasted_iota(jnp.int32, (BLK, RW - 128), 1)
    out_ref[:, 128:RW] = jnp.where(lane == 0, 1.0, 0.0).astype(DT)


def _proj(x, nt2, pg, pbta, pwt, pbi, fg, fb, fwt, fbi, sg, sb, swt, sbi, emb):
    row = lambda i: (i, 0)
    full = lambda i: (0, 0)
    vec = pl.BlockSpec((1, H), full)
    return pl.pallas_call(
        _proj_body,
        grid=(GRID,),
        in_specs=[
            pl.BlockSpec((BLK, D), row),
            pl.BlockSpec((BLK, 1), row),
            vec, vec, pl.BlockSpec((D, H), full), vec,
            vec, vec, pl.BlockSpec((D, H), full), vec,
            vec, vec, pl.BlockSpec((D, H), full), vec,
            pl.BlockSpec((8, H), full),
        ],
        out_specs=pl.BlockSpec((BLK, RW), row),
        out_shape=jax.ShapeDtypeStruct((N, RW), DT),
    )(x, nt2, pg, pbta, pwt, pbi, fg, fb, fwt, fbi, sg, sb, swt, sbi, emb)


def _agg_body(h_hbm, ed_hbm, zero_hbm, out_hbm,
              idx0, idx1, idx2, idx3, rows0, rows1, acc_sh,
              semg0, semg1, semi0, semi1, semi2, semi3):
    c = lax.axis_index("c")
    s = lax.axis_index("s")
    nch = jnp.where(c == 0, N0_CH, N1_CH)
    cbase = c * NS * N0_CH + s * nch
    slots = (idx0, idx1, idx2, idx3)
    isems = (semi0, semi1, semi2, semi3)
    bufs = (rows0, rows1)
    gsems = (semg0, semg1)

    base = s * ROWS_PER_TILE
    pltpu.sync_copy(zero_hbm.at[pl.ds(0, ROWS_PER_TILE)],
                    acc_sh.at[pl.ds(base, ROWS_PER_TILE)])
    plsc.subcore_barrier()

    def start_idx(i, q):
        pltpu.async_copy(ed_hbm.at[cbase + i], slots[q], isems[q])

    def start_gather(i, q, b):
        pltpu.async_copy(h_hbm.at[slots[q].at[0]], bufs[b], gsems[b])

    pltpu.sync_copy(ed_hbm.at[cbase + 0], idx0)
    pltpu.sync_copy(ed_hbm.at[cbase + 1], idx1)
    start_gather(0, 0, 0)
    start_gather(1, 1, 1)
    start_idx(2, 2)
    start_idx(3, 3)

    def body(k, _):
        for b4 in range(4):
            i = 4 * k + b4
            q = b4
            b = b4 % 2
            @pl.when(i < nch)
            def _():
                pltpu.make_async_copy(h_hbm.at[slots[q].at[0]],
                                      bufs[b], gsems[b]).wait()
                pltpu.sync_copy(bufs[b], acc_sh.at[slots[q].at[1]], add=True)
                @pl.when(i + 4 < nch)
                def _():
                    start_idx(i + 4, q)
                @pl.when(i + 2 < nch)
                def _():
                    q2 = (b4 + 2) % 4
                    pltpu.make_async_copy(ed_hbm.at[cbase + i],
                                          slots[q2], isems[q2]).wait()
                    start_gather(i + 2, q2, b)
        return 0
    lax.fori_loop(0, (nch + 3) // 4, body, 0)

    plsc.subcore_barrier()
    pltpu.sync_copy(acc_sh.at[pl.ds(base, ROWS_PER_TILE)],
                    out_hbm.at[c, pl.ds(base, ROWS_PER_TILE)])


def _aggregate(h_aug, ed4, zrows):
    mesh = plsc.VectorSubcoreMesh(core_axis_name="c", subcore_axis_name="s",
                                  num_cores=NC, num_subcores=NS)
    return pl.kernel(
        _agg_body,
        out_type=jax.ShapeDtypeStruct((NC, NP, RW), DT),
        mesh=mesh,
        compiler_params=pltpu.CompilerParams(use_tc_tiling_on_sc=False),
        scratch_types=[
            pltpu.VMEM((2, CHUNK), jnp.int32),
            pltpu.VMEM((2, CHUNK), jnp.int32),
            pltpu.VMEM((2, CHUNK), jnp.int32),
            pltpu.VMEM((2, CHUNK), jnp.int32),
            pltpu.VMEM((CHUNK, RW), DT),
            pltpu.VMEM((CHUNK, RW), DT),
            pltpu.VMEM_SHARED((NP, RW), DT),
            pltpu.SemaphoreType.DMA,
            pltpu.SemaphoreType.DMA,
            pltpu.SemaphoreType.DMA,
            pltpu.SemaphoreType.DMA,
            pltpu.SemaphoreType.DMA,
            pltpu.SemaphoreType.DMA,
        ],
    )(h_aug, ed4, zrows)


def _layer_body(parts_ref, ha_ref, wlt, bl, wrt, g, b, out_ref, *, final):
    pa = parts_ref[0]
    pb = parts_ref[1]
    sums = pa[:, 0:128].astype(jnp.float32) + pb[:, 0:128].astype(jnp.float32)
    cnt = (pa[:, 128:129].astype(jnp.float32)
           + pb[:, 128:129].astype(jnp.float32))
    agg = sums / jnp.maximum(cnt, 1.0)
    h = ha_ref[:, 0:128].astype(jnp.float32)
    t = (jnp.dot(agg, wlt[...], preferred_element_type=jnp.float32)
         + jnp.dot(h, wrt[...], preferred_element_type=jnp.float32)
         + bl[...] + h)
    t = jax.nn.relu(_ln(t, g[...], b[...]))
    if final:
        out_ref[...] = t
    else:
        out_ref[:, 0:128] = t.astype(DT)
        lane = lax.broadcasted_iota(jnp.int32, (BLK, RW - 128), 1)
        out_ref[:, 128:RW] = jnp.where(lane == 0, 1.0, 0.0).astype(DT)


def _layer(parts, h_aug, wlt, bl, wrt, g, b, final):
    row = lambda i: (i, 0)
    full = lambda i: (0, 0)
    vec = pl.BlockSpec((1, H), full)
    ow = H if final else RW
    odt = jnp.float32 if final else DT
    return pl.pallas_call(
        functools.partial(_layer_body, final=final),
        grid=(GRID,),
        in_specs=[
            pl.BlockSpec((NC, BLK, RW), lambda i: (0, i, 0)),
            pl.BlockSpec((BLK, RW), row),
            pl.BlockSpec((H, H), full), vec,
            pl.BlockSpec((H, H), full), vec, vec,
        ],
        out_specs=pl.BlockSpec((BLK, ow), row),
        out_shape=jax.ShapeDtypeStruct((N, ow), odt),
    )(parts, h_aug, wlt, bl, wrt, g, b)


def kernel(x, edge_index, node_type,
           proc_ln_g, proc_ln_b, proc_w, proc_b,
           file_ln_g, file_ln_b, file_w, file_b,
           sock_ln_g, sock_ln_b, sock_w, sock_b,
           type_emb,
           w_l0, b_l0, w_r0, ln_g0, ln_b0,
           w_l1, b_l1, w_r1, ln_g1, ln_b1):
    f32 = jnp.float32
    nt2 = node_type.reshape(N, 1).astype(jnp.int32)
    emb = jnp.zeros((8, H), f32).at[0:3].set(type_emb)
    r1 = lambda v: v.reshape(1, -1).astype(f32)

    h0a = _proj(x, nt2,
                r1(proc_ln_g), r1(proc_ln_b), proc_w.T, r1(proc_b),
                r1(file_ln_g), r1(file_ln_b), file_w.T, r1(file_b),
                r1(sock_ln_g), r1(sock_ln_b), sock_w.T, r1(sock_b),
                emb)

    src = edge_index[0].astype(jnp.int32)
    dst = edge_index[1].astype(jnp.int32)
    pad = E_PAD - E
    src3 = jnp.concatenate([src, jnp.zeros((pad,), jnp.int32)]
                           ).reshape(TOTCH, CHUNK)
    dst3 = jnp.concatenate([dst, jnp.full((pad,), N, jnp.int32)]
                           ).reshape(TOTCH, CHUNK)
    ed4 = jnp.stack([src3, dst3], axis=1)
    zrows = jnp.zeros((ROWS_PER_TILE, RW), DT)

    parts0 = _aggregate(h0a, ed4, zrows)
    h1a = _layer(parts0, h0a, w_l0.T, r1(b_l0), w_r0.T,
                 r1(ln_g0), r1(ln_b0), final=False)
    parts1 = _aggregate(h1a, ed4, zrows)
    h2 = _layer(parts1, h1a, w_l1.T, r1(b_l1), w_r1.T,
                r1(ln_g1), r1(ln_b1), final=True)
    return h2


# speedup vs baseline: 8.1152x; 1.0564x over previous
import functools

import jax
import jax.numpy as jnp
from jax import lax
from jax.experimental import pallas as pl
from jax.experimental.pallas import tpu as pltpu
from jax.experimental.pallas import tpu_sc as plsc

N = 10000
D = 128
H = 128
E = 320000

NC = 2
NS = 16
CHUNK = 128
N0_CH = 111
N1_CH = 46
TOTCH = NS * (N0_CH + N1_CH)
E_PAD = TOTCH * CHUNK
NP = 10016
ROWS_PER_TILE = NP // NS
CW = 16

BLK = 400
GRID = N // BLK


def _ln(h, g, b):
    m = jnp.mean(h, axis=-1, keepdims=True)
    v = jnp.mean((h - m) * (h - m), axis=-1, keepdims=True)
    return (h - m) / jnp.sqrt(v + 1e-5) * g + b


def _proj_body(x_ref, nt_ref, pg, pbta, pwt, pbi, fg, fb, fwt, fbi,
               sg, sb, swt, sbi, emb_ref, out_ref):
    x = jnp.clip(x_ref[...], -10.0, 10.0)
    p = jnp.dot(_ln(x, pg[...], pbta[...]), pwt[...],
                preferred_element_type=jnp.float32) + pbi[...]
    f = jnp.dot(_ln(x, fg[...], fb[...]), fwt[...],
                preferred_element_type=jnp.float32) + fbi[...]
    s = jnp.dot(_ln(x, sg[...], sb[...]), swt[...],
                preferred_element_type=jnp.float32) + sbi[...]
    nt = nt_ref[...]
    sel = jnp.where(nt == 0, p, jnp.where(nt == 1, f,
                    jnp.where(nt == 2, s, 0.0)))
    te = jnp.where(nt == 0, emb_ref[0:1, :], jnp.where(
        nt == 1, emb_ref[1:2, :], emb_ref[2:3, :]))
    out_ref[...] = sel + te


def _proj(x, nt2, pg, pbta, pwt, pbi, fg, fb, fwt, fbi, sg, sb, swt, sbi, emb):
    row = lambda i: (i, 0)
    full = lambda i: (0, 0)
    vec = pl.BlockSpec((1, H), full)
    return pl.pallas_call(
        _proj_body,
        grid=(GRID,),
        in_specs=[
            pl.BlockSpec((BLK, D), row),
            pl.BlockSpec((BLK, 1), row),
            vec, vec, pl.BlockSpec((D, H), full), vec,
            vec, vec, pl.BlockSpec((D, H), full), vec,
            vec, vec, pl.BlockSpec((D, H), full), vec,
            pl.BlockSpec((8, H), full),
        ],
        out_specs=pl.BlockSpec((BLK, H), row),
        out_shape=jax.ShapeDtypeStruct((N, H), jnp.float32),
    )(x, nt2, pg, pbta, pwt, pbi, fg, fb, fwt, fbi, sg, sb, swt, sbi, emb)


def _agg_body(h_hbm, ed_hbm, zero_hbm, out_hbm, cnt_hbm,
              idx0, idx1, idx2, idx3, rows0, rows1, ones_v, acc_sh, cnt_sh,
              semg0, semg1, semi0, semi1, semi2, semi3, semc, *, with_cnt):
    c = lax.axis_index("c")
    s = lax.axis_index("s")
    nch = jnp.where(c == 0, N0_CH, N1_CH)
    cbase = c * NS * N0_CH + s * nch
    slots = (idx0, idx1, idx2, idx3)
    isems = (semi0, semi1, semi2, semi3)
    bufs = (rows0, rows1)
    gsems = (semg0, semg1)

    base = s * ROWS_PER_TILE
    pltpu.sync_copy(zero_hbm.at[pl.ds(0, ROWS_PER_TILE)],
                    acc_sh.at[pl.ds(base, ROWS_PER_TILE)])
    if with_cnt:
        pltpu.sync_copy(zero_hbm.at[pl.ds(0, ROWS_PER_TILE), pl.ds(0, CW)],
                        cnt_sh.at[pl.ds(base, ROWS_PER_TILE)])
        ov = jnp.where(lax.iota(jnp.int32, CW) == 0, 1.0, 0.0)
        def fill(r, _):
            ones_v[r, pl.ds(0, CW)] = ov
            return 0
        lax.fori_loop(0, CHUNK, fill, 0)
    plsc.subcore_barrier()

    def start_idx(i, q):
        pltpu.async_copy(ed_hbm.at[cbase + i], slots[q], isems[q])

    def start_gather(i, q, b):
        pltpu.async_copy(h_hbm.at[slots[q].at[0]], bufs[b], gsems[b])

    pltpu.sync_copy(ed_hbm.at[cbase + 0], idx0)
    pltpu.sync_copy(ed_hbm.at[cbase + 1], idx1)
    start_gather(0, 0, 0)
    start_gather(1, 1, 1)
    start_idx(2, 2)
    start_idx(3, 3)

    def body(k, _):
        for b4 in range(4):
            i = 4 * k + b4
            q = b4
            b = b4 % 2
            @pl.when(i < nch)
            def _():
                pltpu.make_async_copy(h_hbm.at[slots[q].at[0]],
                                      bufs[b], gsems[b]).wait()
                if with_cnt:
                    pltpu.async_copy(ones_v, cnt_sh.at[slots[q].at[1]],
                                     semc, add=True)
                pltpu.sync_copy(bufs[b], acc_sh.at[slots[q].at[1]], add=True)
                if with_cnt:
                    pltpu.make_async_copy(ones_v, cnt_sh.at[slots[q].at[1]],
                                          semc).wait()
                @pl.when(i + 4 < nch)
                def _():
                    start_idx(i + 4, q)
                @pl.when(i + 2 < nch)
                def _():
                    q2 = (b4 + 2) % 4
                    pltpu.make_async_copy(ed_hbm.at[cbase + i],
                                          slots[q2], isems[q2]).wait()
                    start_gather(i + 2, q2, b)
        return 0
    lax.fori_loop(0, (nch + 3) // 4, body, 0)

    plsc.subcore_barrier()
    pltpu.sync_copy(acc_sh.at[pl.ds(base, ROWS_PER_TILE)],
                    out_hbm.at[c, pl.ds(base, ROWS_PER_TILE)])
    if with_cnt:
        pltpu.sync_copy(cnt_sh.at[pl.ds(base, ROWS_PER_TILE)],
                        cnt_hbm.at[c, pl.ds(base, ROWS_PER_TILE)])


def _aggregate(h, ed4, zrows, with_cnt):
    mesh = plsc.VectorSubcoreMesh(core_axis_name="c", subcore_axis_name="s",
                                  num_cores=NC, num_subcores=NS)
    out_type = (jax.ShapeDtypeStruct((NC, NP, H), jnp.float32),
                jax.ShapeDtypeStruct((NC, NP, CW), jnp.float32))
    return pl.kernel(
        functools.partial(_agg_body, with_cnt=with_cnt),
        out_type=out_type,
        mesh=mesh,
        compiler_params=pltpu.CompilerParams(use_tc_tiling_on_sc=False),
        scratch_types=[
            pltpu.VMEM((2, CHUNK), jnp.int32),
            pltpu.VMEM((2, CHUNK), jnp.int32),
            pltpu.VMEM((2, CHUNK), jnp.int32),
            pltpu.VMEM((2, CHUNK), jnp.int32),
            pltpu.VMEM((CHUNK, H), jnp.float32),
            pltpu.VMEM((CHUNK, H), jnp.float32),
            pltpu.VMEM((CHUNK, CW), jnp.float32),
            pltpu.VMEM_SHARED((NP, H), jnp.float32),
            pltpu.VMEM_SHARED((NP, CW), jnp.float32),
            pltpu.SemaphoreType.DMA,
            pltpu.SemaphoreType.DMA,
            pltpu.SemaphoreType.DMA,
            pltpu.SemaphoreType.DMA,
            pltpu.SemaphoreType.DMA,
            pltpu.SemaphoreType.DMA,
            pltpu.SemaphoreType.DMA,
        ],
    )(h, ed4, zrows)


def _layer_body(parts_ref, cnt_ref, h_ref, wlt, bl, wrt, g, b, out_ref):
    sums = parts_ref[0] + parts_ref[1]
    cnt = cnt_ref[0][:, 0:1] + cnt_ref[1][:, 0:1]
    agg = sums / jnp.maximum(cnt, 1.0)
    h = h_ref[...]
    t = (jnp.dot(agg, wlt[...], preferred_element_type=jnp.float32)
         + jnp.dot(h, wrt[...], preferred_element_type=jnp.float32)
         + bl[...] + h)
    out_ref[...] = jax.nn.relu(_ln(t, g[...], b[...]))


def _layer(parts, cnt, h, wlt, bl, wrt, g, b):
    row = lambda i: (i, 0)
    full = lambda i: (0, 0)
    vec = pl.BlockSpec((1, H), full)
    return pl.pallas_call(
        _layer_body,
        grid=(GRID,),
        in_specs=[
            pl.BlockSpec((NC, BLK, H), lambda i: (0, i, 0)),
            pl.BlockSpec((NC, BLK, CW), lambda i: (0, i, 0)),
            pl.BlockSpec((BLK, H), row),
            pl.BlockSpec((H, H), full), vec,
            pl.BlockSpec((H, H), full), vec, vec,
        ],
        out_specs=pl.BlockSpec((BLK, H), row),
        out_shape=jax.ShapeDtypeStruct((N, H), jnp.float32),
    )(parts, cnt, h, wlt, bl, wrt, g, b)


def kernel(x, edge_index, node_type,
           proc_ln_g, proc_ln_b, proc_w, proc_b,
           file_ln_g, file_ln_b, file_w, file_b,
           sock_ln_g, sock_ln_b, sock_w, sock_b,
           type_emb,
           w_l0, b_l0, w_r0, ln_g0, ln_b0,
           w_l1, b_l1, w_r1, ln_g1, ln_b1):
    f32 = jnp.float32
    nt2 = node_type.reshape(N, 1).astype(jnp.int32)
    emb = jnp.zeros((8, H), f32).at[0:3].set(type_emb)
    r1 = lambda v: v.reshape(1, -1).astype(f32)

    h0 = _proj(x, nt2,
               r1(proc_ln_g), r1(proc_ln_b), proc_w.T, r1(proc_b),
               r1(file_ln_g), r1(file_ln_b), file_w.T, r1(file_b),
               r1(sock_ln_g), r1(sock_ln_b), sock_w.T, r1(sock_b),
               emb)

    src = edge_index[0].astype(jnp.int32)
    dst = edge_index[1].astype(jnp.int32)
    pad = E_PAD - E
    src3 = jnp.concatenate([src, jnp.zeros((pad,), jnp.int32)]
                           ).reshape(TOTCH, CHUNK)
    dst3 = jnp.concatenate([dst, jnp.full((pad,), N, jnp.int32)]
                           ).reshape(TOTCH, CHUNK)
    ed4 = jnp.stack([src3, dst3], axis=1)
    zrows = jnp.zeros((ROWS_PER_TILE, H), f32)

    parts0, cnt0 = _aggregate(h0, ed4, zrows, with_cnt=True)
    h1 = _layer(parts0, cnt0, h0, w_l0.T, r1(b_l0), w_r0.T,
                r1(ln_g0), r1(ln_b0))
    parts1, _ = _aggregate(h1, ed4, zrows, with_cnt=False)
    h2 = _layer(parts1, cnt0, h1, w_l1.T, r1(b_l1), w_r1.T,
                r1(ln_g1), r1(ln_b1))
    return h2
